# Initial kernel scaffold; baseline (speedup 1.0000x reference)
import jax
import jax.numpy as jnp
from jax.experimental import pallas as pl


def kernel(unknown, known, unknow_feats, known_feats, W0, gamma0, beta0):
    raise NotImplementedError("write your pallas kernel here")


# speedup vs baseline: 37.6099x; 37.6099x over previous
import functools

import jax
import jax.numpy as jnp
from jax.experimental import pallas as pl

B, N, M, C1, C2, CO = 4, 8192, 2048, 128, 256, 128
NB = 256
NB2 = 2048
_F32_MAX = 3.4e38


def _fp_kernel(unknown_ref, known_ref, uf_ref, kf_ref, w0_ref,
               y_ref, stats_ref):
    b = pl.program_id(0)
    nb = pl.program_id(1)

    u = unknown_ref[0]
    k = known_ref[0]

    d = -2.0 * jax.lax.dot_general(u, k, (((1,), (1,)), ((), ())),
                                   preferred_element_type=jnp.float32)
    d = d + jnp.sum(u * u, axis=1, keepdims=True)
    d = d + jnp.sum(k * k, axis=1)[None, :]

    colid = jax.lax.broadcasted_iota(jnp.int32, (NB, M), 1)

    def pick(dcur):
        mval = jnp.min(dcur, axis=1, keepdims=True)
        midx = jnp.min(jnp.where(dcur == mval, colid, M),
                       axis=1, keepdims=True)
        dnext = jnp.where(colid == midx, _F32_MAX, dcur)
        return mval, midx, dnext

    d1, i1, d_ = pick(d)
    d2, i2, d_ = pick(d_)
    d3, i3, _ = pick(d_)

    w1 = 1.0 / (d1 + 1e-8)
    w2 = 1.0 / (d2 + 1e-8)
    w3 = 1.0 / (d3 + 1e-8)
    ws = w1 + w2 + w3
    w1, w2, w3 = w1 / ws, w2 / ws, w3 / ws

    wsp = jnp.where(colid == i1, w1, 0.0)
    wsp = wsp + jnp.where(colid == i2, w2, 0.0)
    wsp = wsp + jnp.where(colid == i3, w3, 0.0)

    kf = kf_ref[0]
    interp_t = jax.lax.dot_general(kf, wsp, (((1,), (1,)), ((), ())),
                                   preferred_element_type=jnp.float32)

    uf = uf_ref[0]
    w0a = w0_ref[:, :C1]
    w0b = w0_ref[:, C1:]
    y = jnp.dot(w0a, uf, preferred_element_type=jnp.float32)
    y = y + jnp.dot(w0b, interp_t, preferred_element_type=jnp.float32)

    y_ref[0] = y

    @pl.when(jnp.logical_and(b == 0, nb == 0))
    def _():
        stats_ref[...] = jnp.zeros_like(stats_ref)

    ps = jnp.sum(y, axis=1, keepdims=True)
    psq = jnp.sum(y * y, axis=1, keepdims=True)
    pad = jnp.zeros((CO, 126), jnp.float32)
    stats_ref[...] += jnp.concatenate([ps, psq, pad], axis=1)


def _bn_kernel(y_ref, stats_ref, params_ref, out_ref):
    cnt = jnp.float32(B * N)
    mean = stats_ref[:, 0:1] / cnt
    ex2 = stats_ref[:, 1:2] / cnt
    var = ex2 - mean * mean
    rstd = jax.lax.rsqrt(var + 1e-5)
    gamma = params_ref[:, 0:1]
    beta = params_ref[:, 1:2]
    y = y_ref[0]
    out = (y - mean) * (rstd * gamma) + beta
    out_ref[0] = jnp.maximum(out, 0.0)


@jax.jit
def kernel(unknown, known, unknow_feats, known_feats, W0, gamma0, beta0):
    n_blocks = N // NB
    y_raw, stats = pl.pallas_call(
        _fp_kernel,
        grid=(B, n_blocks),
        in_specs=[
            pl.BlockSpec((1, NB, 3), lambda b, n: (b, n, 0)),
            pl.BlockSpec((1, M, 3), lambda b, n: (b, 0, 0)),
            pl.BlockSpec((1, C1, NB), lambda b, n: (b, 0, n)),
            pl.BlockSpec((1, C2, M), lambda b, n: (b, 0, 0)),
            pl.BlockSpec((CO, C1 + C2), lambda b, n: (0, 0)),
        ],
        out_specs=[
            pl.BlockSpec((1, CO, NB), lambda b, n: (b, 0, n)),
            pl.BlockSpec((CO, 128), lambda b, n: (0, 0)),
        ],
        out_shape=[
            jax.ShapeDtypeStruct((B, CO, N), jnp.float32),
            jax.ShapeDtypeStruct((CO, 128), jnp.float32),
        ],
    )(unknown, known, unknow_feats, known_feats, W0)

    params = jnp.zeros((CO, 128), jnp.float32)
    params = params.at[:, 0].set(gamma0).at[:, 1].set(beta0)

    out = pl.pallas_call(
        _bn_kernel,
        grid=(B, N // NB2),
        in_specs=[
            pl.BlockSpec((1, CO, NB2), lambda b, n: (b, 0, n)),
            pl.BlockSpec((CO, 128), lambda b, n: (0, 0)),
            pl.BlockSpec((CO, 128), lambda b, n: (0, 0)),
        ],
        out_specs=pl.BlockSpec((1, CO, NB2), lambda b, n: (b, 0, n)),
        out_shape=jax.ShapeDtypeStruct((B, CO, N), jnp.float32),
    )(y_raw, stats, params)
    return out


# speedup vs baseline: 50.3999x; 1.3401x over previous
import functools

import jax
import jax.numpy as jnp
from jax.experimental import pallas as pl

B, N, M, C1, C2, CO = 4, 8192, 2048, 128, 256, 128
NB = 256
NB2 = 2048
_F32_MAX = 3.4e38


def _fp_kernel(unknown_ref, ka_ref, uf_ref, kf_ref, w0_ref,
               y_ref, stats_ref):
    b = pl.program_id(0)
    nb = pl.program_id(1)

    u = unknown_ref[0]
    k = ka_ref[0][:, :3]

    d = -2.0 * jax.lax.dot_general(u, k, (((1,), (1,)), ((), ())),
                                   preferred_element_type=jnp.float32)
    d = d + jnp.sum(u * u, axis=1, keepdims=True)
    d = d + jnp.sum(k * k, axis=1)[None, :]

    d1 = jnp.min(d, axis=1, keepdims=True)
    dm = jnp.where(d == d1, _F32_MAX, d)
    d2 = jnp.min(dm, axis=1, keepdims=True)
    dm = jnp.where(dm == d2, _F32_MAX, dm)
    d3 = jnp.min(dm, axis=1, keepdims=True)

    w1 = 1.0 / (d1 + 1e-8)
    w2 = 1.0 / (d2 + 1e-8)
    w3 = 1.0 / (d3 + 1e-8)
    ws = w1 + w2 + w3
    w1, w2, w3 = w1 / ws, w2 / ws, w3 / ws

    wsp = jnp.where(d == d1, w1,
                    jnp.where(d == d2, w2,
                              jnp.where(d == d3, w3, 0.0)))

    kf = kf_ref[0]
    interp_t = jax.lax.dot_general(kf, wsp, (((1,), (1,)), ((), ())),
                                   preferred_element_type=jnp.float32)

    uf = uf_ref[0]
    w0a = w0_ref[:, :C1]
    w0b = w0_ref[:, C1:]
    y = jnp.dot(w0a, uf, preferred_element_type=jnp.float32)
    y = y + jnp.dot(w0b, interp_t, preferred_element_type=jnp.float32)

    y_ref[0] = y

    @pl.when(jnp.logical_and(b == 0, nb == 0))
    def _():
        stats_ref[...] = jnp.zeros_like(stats_ref)

    ps = jnp.sum(y, axis=1, keepdims=True)
    psq = jnp.sum(y * y, axis=1, keepdims=True)
    pad = jnp.zeros((CO, 126), jnp.float32)
    stats_ref[...] += jnp.concatenate([ps, psq, pad], axis=1)


def _bn_kernel(y_ref, stats_ref, params_ref, out_ref):
    cnt = jnp.float32(B * N)
    mean = stats_ref[:, 0:1] / cnt
    ex2 = stats_ref[:, 1:2] / cnt
    var = ex2 - mean * mean
    rstd = jax.lax.rsqrt(var + 1e-5)
    gamma = params_ref[:, 0:1]
    beta = params_ref[:, 1:2]
    y = y_ref[0]
    out = (y - mean) * (rstd * gamma) + beta
    out_ref[0] = jnp.maximum(out, 0.0)


@jax.jit
def kernel(unknown, known, unknow_feats, known_feats, W0, gamma0, beta0):
    n_blocks = N // NB
    known_aug = jnp.concatenate(
        [known, jnp.sum(known * known, axis=2, keepdims=True)], axis=2)
    y_raw, stats = pl.pallas_call(
        _fp_kernel,
        grid=(B, n_blocks),
        in_specs=[
            pl.BlockSpec((1, NB, 3), lambda b, n: (b, n, 0)),
            pl.BlockSpec((1, M, 4), lambda b, n: (b, 0, 0)),
            pl.BlockSpec((1, C1, NB), lambda b, n: (b, 0, n)),
            pl.BlockSpec((1, C2, M), lambda b, n: (b, 0, 0)),
            pl.BlockSpec((CO, C1 + C2), lambda b, n: (0, 0)),
        ],
        out_specs=[
            pl.BlockSpec((1, CO, NB), lambda b, n: (b, 0, n)),
            pl.BlockSpec((CO, 128), lambda b, n: (0, 0)),
        ],
        out_shape=[
            jax.ShapeDtypeStruct((B, CO, N), jnp.float32),
            jax.ShapeDtypeStruct((CO, 128), jnp.float32),
        ],
    )(unknown, known_aug, unknow_feats, known_feats, W0)

    params = jnp.zeros((CO, 128), jnp.float32)
    params = params.at[:, 0].set(gamma0).at[:, 1].set(beta0)

    out = pl.pallas_call(
        _bn_kernel,
        grid=(B, N // NB2),
        in_specs=[
            pl.BlockSpec((1, CO, NB2), lambda b, n: (b, 0, n)),
            pl.BlockSpec((CO, 128), lambda b, n: (0, 0)),
            pl.BlockSpec((CO, 128), lambda b, n: (0, 0)),
        ],
        out_specs=pl.BlockSpec((1, CO, NB2), lambda b, n: (b, 0, n)),
        out_shape=jax.ShapeDtypeStruct((B, CO, N), jnp.float32),
    )(y_raw, stats, params)
    return out
